# baseline (speedup 1.0000x reference)
import functools

import jax
import jax.numpy as jnp
from jax import lax
from jax.experimental import pallas as pl
from jax.experimental.pallas import tpu as pltpu
from jax.experimental.pallas import tpu_sc as plsc

E = 320000
D = 128
NC = 2
NS = 16
NW = NC * NS
EPW = E // NW
C = 80
NCHUNK = EPW // C

_mesh = plsc.VectorSubcoreMesh(core_axis_name="c", subcore_axis_name="s")


@functools.partial(
    pl.kernel,
    out_type=jax.ShapeDtypeStruct((E,), jnp.float32),
    mesh=_mesh,
    scratch_types=[
        pltpu.VMEM((C,), jnp.int32),
        pltpu.VMEM((C,), jnp.int32),
        pltpu.VMEM((C,), jnp.int32),
        pltpu.VMEM((C,), jnp.int32),
        pltpu.VMEM((C, D), jnp.float32),
        pltpu.VMEM((C, D), jnp.float32),
        pltpu.VMEM((C, D), jnp.float32),
        pltpu.VMEM((C, D), jnp.float32),
        pltpu.VMEM((C,), jnp.float32),
        pltpu.VMEM((C,), jnp.float32),
        pltpu.SemaphoreType.DMA,
        pltpu.SemaphoreType.DMA,
    ],
    compiler_params=pltpu.CompilerParams(needs_layout_passes=False),
)
def _ip_decode(z_hbm, row_hbm, col_hbm, out_hbm,
               ri0, ri1, ci0, ci1, a0, a1, b0, b1, o0, o1, s0, s1):
    wid = lax.axis_index("s") * NC + lax.axis_index("c")
    ebase = wid * EPW

    ab = ((ri0, ci0, a0, b0, o0, s0), (ri1, ci1, a1, b1, o1, s1))
    lane = lax.iota(jnp.int32, 16)

    def issue(ci, slot):
        ri, cidx, a, b, _, sem = ab[slot]
        off = ebase + ci * C
        pltpu.sync_copy(row_hbm.at[pl.ds(off, C)], ri)
        pltpu.sync_copy(col_hbm.at[pl.ds(off, C)], cidx)
        pltpu.async_copy(z_hbm.at[ri], a, sem)
        pltpu.async_copy(z_hbm.at[cidx], b, sem)

    def drain(slot):
        _, _, a, b, _, sem = ab[slot]
        pltpu.make_async_copy(z_hbm.at[pl.ds(0, C)], a, sem).wait()
        pltpu.make_async_copy(z_hbm.at[pl.ds(0, C)], b, sem).wait()

    def compute(ci, slot):
        _, _, a, b, o, _ = ab[slot]

        def group_body(g, carry2):
            res = jnp.zeros((16,), jnp.float32)
            for j in range(16):
                e = g * 16 + j
                acc = jnp.zeros((16,), jnp.float32)
                for k in range(D // 16):
                    acc = acc + (a[e, pl.ds(k * 16, 16)] *
                                 b[e, pl.ds(k * 16, 16)])
                res = jnp.where(lane == j, jnp.sum(acc), res)
            o[pl.ds(g * 16, 16)] = res
            return carry2

        lax.fori_loop(0, C // 16, group_body, 0)
        pltpu.sync_copy(o, out_hbm.at[pl.ds(ebase + ci * C, C)])

    issue(0, 0)
    issue(1, 1)

    def chunk_pair(i, carry):
        for j in range(2):
            ci = i * 2 + j
            drain(j)
            compute(ci, j)
            pl.when(ci + 2 < NCHUNK)(lambda: issue(ci + 2, j))
        return carry

    lax.fori_loop(0, (NCHUNK - 1) // 2, chunk_pair, 0)
    drain(0)
    compute(NCHUNK - 1, 0)


def kernel(z, edge_index):
    row = edge_index[0].astype(jnp.int32)
    col = edge_index[1].astype(jnp.int32)
    return _ip_decode(z, row, col)
